# baseline (speedup 1.0000x reference)
import jax
import jax.numpy as jnp
from jax import lax
from jax.experimental import pallas as pl
from jax.experimental.pallas import tpu as pltpu

N = 10000
D_IN = 128
D_OUT = 128
CB = 200
NCHUNK = N // CB
BM = 2 * CB
NPAIR = N // BM


def _chunk_copy(adj_ref, buf_ref, chunk, slot, sem):
    return pltpu.make_async_copy(adj_ref.at[chunk], buf_ref.at[slot], sem)


def _pair_wait(adj_ref, buf_ref, pair, sel, sem):
    return pltpu.make_async_copy(
        adj_ref.at[pl.ds(2 * pair, 2)], buf_ref.at[pl.ds(sel, 2)], sem
    )


def _out_copy(ostg_ref, out_ref, pair, oslot, out_sems):
    return pltpu.make_async_copy(
        ostg_ref.at[oslot], out_ref.at[pl.ds(pair * BM, BM), :], out_sems.at[oslot]
    )


def _gcn_kernel(x_ref, w_ref, adj_ref, out_ref,
                xv_ref, support_ref, buf_ref, ostg_ref,
                x_sem, in_sems, out_sems):
    x_copy = pltpu.make_async_copy(x_ref, xv_ref, x_sem)
    x_copy.start()
    for k in range(4):
        _chunk_copy(adj_ref, buf_ref, k, k, in_sems.at[(k // 2) % 2]).start()
    x_copy.wait()
    support_ref[...] = jnp.dot(
        xv_ref[...], w_ref[...], preferred_element_type=jnp.float32
    )

    def body(c, _):
        par = lax.rem(c, 2)
        sel = par * 2
        _pair_wait(adj_ref, buf_ref, c, sel, in_sems.at[par]).wait()
        pair_block = buf_ref[pl.ds(sel, 2)].reshape(BM, N)
        result = jnp.maximum(
            jnp.dot(pair_block, support_ref[...],
                    preferred_element_type=jnp.float32),
            0.0,
        )

        @pl.when(c >= 2)
        def _():
            _out_copy(ostg_ref, out_ref, c - 2, par, out_sems).wait()

        ostg_ref[par] = result
        _out_copy(ostg_ref, out_ref, c, par, out_sems).start()

        @pl.when(c + 2 < NPAIR)
        def _():
            _chunk_copy(adj_ref, buf_ref, 2 * c + 4, sel, in_sems.at[par]).start()
            _chunk_copy(adj_ref, buf_ref, 2 * c + 5, sel + 1, in_sems.at[par]).start()

        return _

    lax.fori_loop(0, NPAIR, body, None)

    for p in range(NPAIR - 2, NPAIR):
        _out_copy(ostg_ref, out_ref, p, p % 2, out_sems).wait()


def kernel(input, adj, weight):
    adj3 = adj.reshape(NCHUNK, CB, N)
    return pl.pallas_call(
        _gcn_kernel,
        in_specs=[
            pl.BlockSpec(memory_space=pltpu.MemorySpace.HBM),
            pl.BlockSpec((D_IN, D_OUT), lambda: (0, 0)),
            pl.BlockSpec(memory_space=pltpu.MemorySpace.HBM),
        ],
        out_specs=pl.BlockSpec(memory_space=pltpu.MemorySpace.HBM),
        out_shape=jax.ShapeDtypeStruct((N, D_OUT), jnp.float32),
        scratch_shapes=[
            pltpu.VMEM((N, D_IN), jnp.float32),
            pltpu.VMEM((N, D_OUT), jnp.float32),
            pltpu.VMEM((4, CB, N), jnp.float32),
            pltpu.VMEM((2, BM, D_OUT), jnp.float32),
            pltpu.SemaphoreType.DMA,
            pltpu.SemaphoreType.DMA((2,)),
            pltpu.SemaphoreType.DMA((2,)),
        ],
    )(input, weight, adj3)


# speedup vs baseline: 1.0177x; 1.0177x over previous
import jax
import jax.numpy as jnp
from jax.experimental import pallas as pl
from jax.experimental.pallas import tpu as pltpu

N = 10000
D_IN = 128
D_OUT = 128
BM = 400


def _gcn_kernel(x_ref, w_ref, adj_ref, out_ref, support_ref):
    @pl.when(pl.program_id(0) == 0)
    def _():
        support_ref[...] = jnp.dot(
            x_ref[...], w_ref[...], preferred_element_type=jnp.float32
        )

    acc = jnp.dot(
        adj_ref[...], support_ref[...], preferred_element_type=jnp.float32
    )
    out_ref[...] = jnp.maximum(acc, 0.0)


def kernel(input, adj, weight):
    grid = (pl.cdiv(N, BM),)
    return pl.pallas_call(
        _gcn_kernel,
        grid=grid,
        in_specs=[
            pl.BlockSpec((N, D_IN), lambda i: (0, 0)),
            pl.BlockSpec((D_IN, D_OUT), lambda i: (0, 0)),
            pl.BlockSpec((BM, N), lambda i: (i, 0)),
        ],
        out_specs=pl.BlockSpec((BM, D_OUT), lambda i: (i, 0)),
        out_shape=jax.ShapeDtypeStruct((N, D_OUT), jnp.float32),
        scratch_shapes=[pltpu.VMEM((N, D_OUT), jnp.float32)],
        compiler_params=pltpu.CompilerParams(
            dimension_semantics=("arbitrary",),
        ),
    )(input, weight, adj)
